# baseline (speedup 1.0000x reference)
import functools

import jax
import jax.numpy as jnp
from jax import lax
from jax.experimental import pallas as pl
from jax.experimental.pallas import tpu as pltpu
from jax.experimental.pallas import tpu_sc as plsc

N = 10000
D_IN = 256
D_OUT = 256
DH = 128
NEG_SLOPE = 0.2

NC = 2
NS = 16
L = 16

ET = 160000 + N
K = 64
NB = 4
_c = -(-(-(-ET // NS)) // K)
C_CHUNKS = -(-_c // NB) * NB
PT = C_CHUNKS * K
E_PAD = PT * NS

ROW_CHUNKS = -(-N // K)
LAST_ROWS = N - (ROW_CHUNKS - 1) * K


def _tc_body(nf_ref, w_ref, attp_ref, x0_ref, x1_ref, att_ref):
    xw = jnp.dot(nf_ref[...], w_ref[...], preferred_element_type=jnp.float32)
    x0_ref[...] = xw[:, :DH]
    x1_ref[...] = xw[:, DH:]
    att_ref[...] = lax.dot_general(
        attp_ref[...], xw, (((0,), (1,)), ((), ())),
        preferred_element_type=jnp.float32)


def _dense_stage(node_feature, W, attp):
    return pl.pallas_call(
        _tc_body,
        out_shape=[
            jax.ShapeDtypeStruct((N, DH), jnp.float32),
            jax.ShapeDtypeStruct((N, DH), jnp.float32),
            jax.ShapeDtypeStruct((8, N), jnp.float32),
        ],
    )(node_feature, W, attp)


def _s_body(att_hbm, src_hbm, dst_hbm, s_hbm, table, idxb, s_all):
    cid = lax.axis_index("c")
    sid = lax.axis_index("s")

    @pl.when(cid == 0)
    def _():
        base = sid * PT
        iota = lax.iota(jnp.int32, L)

        pltpu.sync_copy(att_hbm.at[0], table)
        pltpu.sync_copy(src_hbm.at[sid], idxb)

        def p1(c, _):
            for q in range(K // L):
                si = idxb[c, pl.ds(q * L, L)]
                s_all[c, pl.ds(q * L, L)] = plsc.load_gather(table, [si])
            return 0
        lax.fori_loop(0, C_CHUNKS, p1, 0, unroll=2)

        pltpu.sync_copy(att_hbm.at[1], table)
        pltpu.sync_copy(dst_hbm.at[sid], idxb)

        def p2(c, _):
            for q in range(K // L):
                di = idxb[c, pl.ds(q * L, L)]
                al = s_all[c, pl.ds(q * L, L)] + plsc.load_gather(table, [di])
                al = jnp.where(al > 0, al, al * NEG_SLOPE)
                s = jnp.exp(al)
                pos = base + c * K + q * L + iota
                s_all[c, pl.ds(q * L, L)] = jnp.where(pos < ET, s, 0.0)
            return 0
        lax.fori_loop(0, C_CHUNKS, p2, 0, unroll=2)

        pltpu.sync_copy(s_all, s_hbm.at[sid])


_s_stage = functools.partial(
    pl.kernel,
    out_type=jax.ShapeDtypeStruct((NS, C_CHUNKS, K), jnp.float32),
    mesh=plsc.VectorSubcoreMesh(core_axis_name="c", subcore_axis_name="s",
                                num_cores=NC, num_subcores=NS),
    compiler_params=pltpu.CompilerParams(needs_layout_passes=False),
    scratch_types=[
        pltpu.VMEM((N,), jnp.float32),
        pltpu.VMEM((C_CHUNKS, K), jnp.int32),
        pltpu.VMEM((C_CHUNKS, K), jnp.float32),
    ],
)(_s_body)


def _scale_rows(buf, b, sbuf, n_rows):
    bsplat = jnp.full((L,), b, jnp.int32)

    def body(i, _):
        sv = plsc.load_gather(sbuf, [bsplat, jnp.full((L,), i, jnp.int32)])
        for q in range(DH // L):
            buf[b, i, pl.ds(q * L, L)] = buf[b, i, pl.ds(q * L, L)] * sv
        return 0
    lax.fori_loop(0, n_rows, body, 0, unroll=2)


def _unpack_scale_rows(gbuf, sstg, b, sbuf, n_rows):
    bsplat = jnp.full((L,), b, jnp.int32)

    def body(i, _):
        sv = plsc.load_gather(sbuf, [bsplat, jnp.full((L,), i, jnp.int32)])
        for m in range(DH // 32):
            v = gbuf[b, i, pl.ds(m * L, L)]
            bfv = plsc.bitcast(v, jnp.bfloat16)
            lo, hi = plsc.unpack(bfv, format=plsc.PackFormat.INTERLEAVED)
            sstg[b, i, pl.ds(m * 32, L)] = lo * sv
            sstg[b, i, pl.ds(m * 32 + L, L)] = hi * sv
        return 0
    lax.fori_loop(0, n_rows, body, 0, unroll=2)


def _agg_body(x0_hbm, x1_hbm, src_hbm, dst_hbm, s_hbm, out_hbm,
              srcb, dstb, sbuf, gbuf, sstg, acc_sp, den_sp,
              sem_g0, sem_g1, sem_g2, sem_g3,
              sem_sc0, sem_sc1, sem_sc2, sem_sc3,
              sem_s0, sem_s1, sem_s2, sem_s3,
              sem_d0, sem_d1, sem_d2, sem_d3):
    cid = lax.axis_index("c")
    sid = lax.axis_index("s")
    sem_g = (sem_g0, sem_g1, sem_g2, sem_g3)
    sem_sc = (sem_sc0, sem_sc1, sem_sc2, sem_sc3)
    sem_s = (sem_s0, sem_s1, sem_s2, sem_s3)
    sem_d = (sem_d0, sem_d1, sem_d2, sem_d3)

    def zrow(i, _):
        for q in range(DH // L):
            sstg[0, i, pl.ds(q * L, L)] = jnp.zeros((L,), jnp.float32)
        return 0
    lax.fori_loop(0, K, zrow, 0)
    for q in range(K // L):
        sbuf[0, pl.ds(q * L, L)] = jnp.zeros((L,), jnp.float32)
    for j in range(-(-ROW_CHUNKS // NS)):
        c = sid + NS * j

        @pl.when(c < ROW_CHUNKS - 1)
        def _():
            pltpu.sync_copy(sstg.at[0], acc_sp.at[pl.ds(c * K, K), :])
            pltpu.sync_copy(sbuf.at[0], den_sp.at[pl.ds(c * K, K)])

        @pl.when(c == ROW_CHUNKS - 1)
        def _():
            pltpu.sync_copy(sstg.at[0, pl.ds(0, LAST_ROWS), :],
                            acc_sp.at[pl.ds((ROW_CHUNKS - 1) * K, LAST_ROWS), :])
            pltpu.sync_copy(sbuf.at[0, pl.ds(0, LAST_ROWS)],
                            den_sp.at[pl.ds((ROW_CHUNKS - 1) * K, LAST_ROWS)])
    plsc.subcore_barrier()

    def fire_src(c, m):
        pltpu.async_copy(src_hbm.at[sid, c], srcb.at[m], sem_s[m])

    def wait_src(c, m):
        pltpu.make_async_copy(src_hbm.at[sid, c], srcb.at[m], sem_s[m]).wait()

    def fire_ds(c, m):
        pltpu.async_copy(dst_hbm.at[sid, c], dstb.at[m], sem_d[m])
        pltpu.async_copy(s_hbm.at[sid, c], sbuf.at[m], sem_d[m])

    def wait_ds(c, m):
        pltpu.make_async_copy(dst_hbm.at[sid, c], dstb.at[m], sem_d[m]).wait()
        pltpu.make_async_copy(s_hbm.at[sid, c], sbuf.at[m], sem_d[m]).wait()

    def fire_gather(m):
        @pl.when(cid == 0)
        def _():
            pltpu.async_copy(x0_hbm.at[srcb.at[m]], gbuf.at[m], sem_g[m])

        @pl.when(cid == 1)
        def _():
            pltpu.async_copy(x1_hbm.at[srcb.at[m]], gbuf.at[m], sem_g[m])

    def wait_gather(m):
        pltpu.make_async_copy(x0_hbm.at[srcb.at[m]], gbuf.at[m],
                              sem_g[m]).wait()

    def fire_scatter(m):
        pltpu.async_copy(sstg.at[m], acc_sp.at[dstb.at[m]], sem_sc[m],
                         add=True)
        pltpu.async_copy(sbuf.at[m], den_sp.at[dstb.at[m]], sem_sc[m],
                         add=True)

    def wait_scatter(m):
        pltpu.make_async_copy(sstg.at[m], acc_sp.at[dstb.at[m]],
                              sem_sc[m]).wait()
        pltpu.make_async_copy(sbuf.at[m], den_sp.at[dstb.at[m]],
                              sem_sc[m]).wait()

    pltpu.sync_copy(src_hbm.at[sid, 0], srcb.at[0])
    pltpu.sync_copy(src_hbm.at[sid, 1], srcb.at[1])
    pltpu.sync_copy(src_hbm.at[sid, 2], srcb.at[2])
    for m in range(NB):
        pltpu.sync_copy(dst_hbm.at[sid, m], dstb.at[m])
        pltpu.sync_copy(s_hbm.at[sid, m], sbuf.at[m])
    fire_gather(0)
    fire_gather(1)

    def quad(p, _):
        for b in range(NB):
            c = NB * p + b
            m1 = (b + 1) % NB
            m2 = (b + 2) % NB
            m3 = (b + 3) % NB

            @pl.when(c >= 2)
            def _():
                wait_scatter(m2)

            @pl.when(jnp.logical_and(c >= 2, c + 2 < C_CHUNKS))
            def _():
                fire_ds(c + 2, m2)

            @pl.when(c + 3 < C_CHUNKS)
            def _():
                fire_src(c + 3, m3)

            @pl.when(jnp.logical_and(c >= 1, c + 2 < C_CHUNKS))
            def _():
                wait_src(c + 2, m2)

            @pl.when(c + 2 < C_CHUNKS)
            def _():
                fire_gather(m2)

            wait_gather(b)

            @pl.when(c >= 4)
            def _():
                wait_ds(c, b)
            _unpack_scale_rows(gbuf, sstg, b, sbuf, K)
            fire_scatter(b)
        return 0

    lax.fori_loop(0, C_CHUNKS // NB, quad, 0)
    wait_scatter((C_CHUNKS - 2) % NB)
    wait_scatter((C_CHUNKS - 1) % NB)
    plsc.subcore_barrier()

    def norm_write(c, n_rows):
        pltpu.sync_copy(acc_sp.at[pl.ds(c * K, n_rows), :],
                        sstg.at[0, pl.ds(0, n_rows), :])
        pltpu.sync_copy(den_sp.at[pl.ds(c * K, n_rows)],
                        sbuf.at[0, pl.ds(0, n_rows)])
        for q in range(n_rows // L):
            sbuf[0, pl.ds(q * L, L)] = 1.0 / sbuf[0, pl.ds(q * L, L)]
        _scale_rows(sstg, 0, sbuf, n_rows)

        @pl.when(cid == 0)
        def _():
            pltpu.sync_copy(sstg.at[0, pl.ds(0, n_rows), :],
                            out_hbm.at[pl.ds(c * K, n_rows), pl.ds(0, DH)])

        @pl.when(cid == 1)
        def _():
            pltpu.sync_copy(sstg.at[0, pl.ds(0, n_rows), :],
                            out_hbm.at[pl.ds(c * K, n_rows), pl.ds(DH, DH)])

    for j in range(-(-ROW_CHUNKS // NS)):
        c = sid + NS * j

        @pl.when(c < ROW_CHUNKS - 1)
        def _():
            norm_write(c, K)

        @pl.when(c == ROW_CHUNKS - 1)
        def _():
            norm_write(jnp.int32(ROW_CHUNKS - 1), LAST_ROWS)


_agg_stage = functools.partial(
    pl.kernel,
    out_type=jax.ShapeDtypeStruct((N, D_OUT), jnp.float32),
    mesh=plsc.VectorSubcoreMesh(core_axis_name="c", subcore_axis_name="s",
                                num_cores=NC, num_subcores=NS),
    compiler_params=pltpu.CompilerParams(needs_layout_passes=False,
                                         use_tc_tiling_on_sc=False),
    scratch_types=(
        [
            pltpu.VMEM((NB, K), jnp.int32),
            pltpu.VMEM((NB, K), jnp.int32),
            pltpu.VMEM((NB, K), jnp.float32),
            pltpu.VMEM((NB, K, DH // 2), jnp.int32),
            pltpu.VMEM((NB, K, DH), jnp.float32),
            pltpu.VMEM_SHARED((N, DH), jnp.float32),
            pltpu.VMEM_SHARED((N,), jnp.float32),
        ]
        + [pltpu.SemaphoreType.DMA] * 16
    ),
)(_agg_body)


def kernel(node_feature, edge_index, W, att_src, att_dst):
    attp = jnp.concatenate(
        [att_src[:, None], att_dst[:, None],
         jnp.zeros((D_IN, 6), jnp.float32)], axis=1)
    x0, x1, att = _dense_stage(node_feature, W, attp)

    ei = edge_index.astype(jnp.int32)
    loops = jnp.arange(N, dtype=jnp.int32)
    src = jnp.concatenate([ei[0], loops])
    dst = jnp.concatenate([ei[1], loops])
    src = jnp.pad(src, (0, E_PAD - ET)).reshape(NS, C_CHUNKS, K)
    dst = jnp.pad(dst, (0, E_PAD - ET)).reshape(NS, C_CHUNKS, K)

    def _pack(xh):
        xbf = xh.astype(jnp.bfloat16).reshape(N, 4, 2, 16)
        xbf = xbf.transpose(0, 1, 3, 2).reshape(N, DH // 2, 2)
        return lax.bitcast_convert_type(xbf, jnp.int32)

    s_pad = _s_stage(att, src, dst)
    return _agg_stage(_pack(x0), _pack(x1), src, dst, s_pad)


# speedup vs baseline: 1.0009x; 1.0009x over previous
import functools

import jax
import jax.numpy as jnp
from jax import lax
from jax.experimental import pallas as pl
from jax.experimental.pallas import tpu as pltpu
from jax.experimental.pallas import tpu_sc as plsc

N = 10000
D_IN = 256
D_OUT = 256
DH = 128
NEG_SLOPE = 0.2

NC = 2
NS = 16
L = 16

ET = 160000 + N
K = 64
NB = 4
_c = -(-(-(-ET // NS)) // K)
C_CHUNKS = -(-_c // NB) * NB
PT = C_CHUNKS * K
E_PAD = PT * NS

ROW_CHUNKS = -(-N // K)
LAST_ROWS = N - (ROW_CHUNKS - 1) * K


def _tc_body(nf_ref, w_ref, attp_ref, x0_ref, x1_ref, att_ref):
    xw = jnp.dot(nf_ref[...], w_ref[...], preferred_element_type=jnp.float32)
    x0_ref[...] = xw[:, :DH]
    x1_ref[...] = xw[:, DH:]
    att_ref[...] = lax.dot_general(
        attp_ref[...], xw, (((0,), (1,)), ((), ())),
        preferred_element_type=jnp.float32)


def _dense_stage(node_feature, W, attp):
    return pl.pallas_call(
        _tc_body,
        out_shape=[
            jax.ShapeDtypeStruct((N, DH), jnp.float32),
            jax.ShapeDtypeStruct((N, DH), jnp.float32),
            jax.ShapeDtypeStruct((8, N), jnp.float32),
        ],
    )(node_feature, W, attp)


def _s_body(att_hbm, src_hbm, dst_hbm, s_hbm, table, idxb, s_all):
    cid = lax.axis_index("c")
    sid = lax.axis_index("s")

    @pl.when(cid == 0)
    def _():
        base = sid * PT
        iota = lax.iota(jnp.int32, L)

        pltpu.sync_copy(att_hbm.at[0], table)
        pltpu.sync_copy(src_hbm.at[sid], idxb)

        def p1(c, _):
            for q in range(K // L):
                si = idxb[c, pl.ds(q * L, L)]
                s_all[c, pl.ds(q * L, L)] = plsc.load_gather(table, [si])
            return 0
        lax.fori_loop(0, C_CHUNKS, p1, 0, unroll=2)

        pltpu.sync_copy(att_hbm.at[1], table)
        pltpu.sync_copy(dst_hbm.at[sid], idxb)

        def p2(c, _):
            for q in range(K // L):
                di = idxb[c, pl.ds(q * L, L)]
                al = s_all[c, pl.ds(q * L, L)] + plsc.load_gather(table, [di])
                al = jnp.where(al > 0, al, al * NEG_SLOPE)
                s = jnp.exp(al)
                pos = base + c * K + q * L + iota
                s_all[c, pl.ds(q * L, L)] = jnp.where(pos < ET, s, 0.0)
            return 0
        lax.fori_loop(0, C_CHUNKS, p2, 0, unroll=2)

        pltpu.sync_copy(s_all, s_hbm.at[sid])


_s_stage = functools.partial(
    pl.kernel,
    out_type=jax.ShapeDtypeStruct((NS, C_CHUNKS, K), jnp.float32),
    mesh=plsc.VectorSubcoreMesh(core_axis_name="c", subcore_axis_name="s",
                                num_cores=NC, num_subcores=NS),
    compiler_params=pltpu.CompilerParams(needs_layout_passes=False),
    scratch_types=[
        pltpu.VMEM((N,), jnp.float32),
        pltpu.VMEM((C_CHUNKS, K), jnp.int32),
        pltpu.VMEM((C_CHUNKS, K), jnp.float32),
    ],
)(_s_body)


def _scale_rows(buf, b, sbuf, n_rows):
    bsplat = jnp.full((L,), b, jnp.int32)

    def body(i, _):
        sv = plsc.load_gather(sbuf, [bsplat, jnp.full((L,), i, jnp.int32)])
        for q in range(DH // L):
            buf[b, i, pl.ds(q * L, L)] = buf[b, i, pl.ds(q * L, L)] * sv
        return 0
    lax.fori_loop(0, n_rows, body, 0, unroll=2)


def _unpack_scale_rows(gbuf, sstg, b, sbuf, n_rows):
    bsplat = jnp.full((L,), b, jnp.int32)

    himask = jnp.full((L,), -65536, jnp.int32)

    def body(i, _):
        sv = plsc.load_gather(sbuf, [bsplat, jnp.full((L,), i, jnp.int32)])
        for m in range(DH // 32):
            v = gbuf[b, i, pl.ds(m * L, L)]
            lo = plsc.bitcast(jnp.left_shift(v, 16), jnp.float32)
            hi = plsc.bitcast(jnp.bitwise_and(v, himask), jnp.float32)
            sstg[b, i, pl.ds(m * 32, L)] = lo * sv
            sstg[b, i, pl.ds(m * 32 + L, L)] = hi * sv
        return 0
    lax.fori_loop(0, n_rows, body, 0, unroll=2)


def _agg_body(x0_hbm, x1_hbm, src_hbm, dst_hbm, s_hbm, out_hbm,
              srcb, dstb, sbuf, gbuf, sstg, acc_sp, den_sp,
              sem_g0, sem_g1, sem_g2, sem_g3,
              sem_sc0, sem_sc1, sem_sc2, sem_sc3,
              sem_s0, sem_s1, sem_s2, sem_s3,
              sem_d0, sem_d1, sem_d2, sem_d3):
    cid = lax.axis_index("c")
    sid = lax.axis_index("s")
    sem_g = (sem_g0, sem_g1, sem_g2, sem_g3)
    sem_sc = (sem_sc0, sem_sc1, sem_sc2, sem_sc3)
    sem_s = (sem_s0, sem_s1, sem_s2, sem_s3)
    sem_d = (sem_d0, sem_d1, sem_d2, sem_d3)

    def zrow(i, _):
        for q in range(DH // L):
            sstg[0, i, pl.ds(q * L, L)] = jnp.zeros((L,), jnp.float32)
        return 0
    lax.fori_loop(0, K, zrow, 0)
    for q in range(K // L):
        sbuf[0, pl.ds(q * L, L)] = jnp.zeros((L,), jnp.float32)
    for j in range(-(-ROW_CHUNKS // NS)):
        c = sid + NS * j

        @pl.when(c < ROW_CHUNKS - 1)
        def _():
            pltpu.sync_copy(sstg.at[0], acc_sp.at[pl.ds(c * K, K), :])
            pltpu.sync_copy(sbuf.at[0], den_sp.at[pl.ds(c * K, K)])

        @pl.when(c == ROW_CHUNKS - 1)
        def _():
            pltpu.sync_copy(sstg.at[0, pl.ds(0, LAST_ROWS), :],
                            acc_sp.at[pl.ds((ROW_CHUNKS - 1) * K, LAST_ROWS), :])
            pltpu.sync_copy(sbuf.at[0, pl.ds(0, LAST_ROWS)],
                            den_sp.at[pl.ds((ROW_CHUNKS - 1) * K, LAST_ROWS)])
    plsc.subcore_barrier()

    def fire_src(c, m):
        pltpu.async_copy(src_hbm.at[sid, c], srcb.at[m], sem_s[m])

    def wait_src(c, m):
        pltpu.make_async_copy(src_hbm.at[sid, c], srcb.at[m], sem_s[m]).wait()

    def fire_ds(c, m):
        pltpu.async_copy(dst_hbm.at[sid, c], dstb.at[m], sem_d[m])
        pltpu.async_copy(s_hbm.at[sid, c], sbuf.at[m], sem_d[m])

    def wait_ds(c, m):
        pltpu.make_async_copy(dst_hbm.at[sid, c], dstb.at[m], sem_d[m]).wait()
        pltpu.make_async_copy(s_hbm.at[sid, c], sbuf.at[m], sem_d[m]).wait()

    def fire_gather(m):
        @pl.when(cid == 0)
        def _():
            pltpu.async_copy(x0_hbm.at[srcb.at[m]], gbuf.at[m], sem_g[m])

        @pl.when(cid == 1)
        def _():
            pltpu.async_copy(x1_hbm.at[srcb.at[m]], gbuf.at[m], sem_g[m])

    def wait_gather(m):
        pltpu.make_async_copy(x0_hbm.at[srcb.at[m]], gbuf.at[m],
                              sem_g[m]).wait()

    def fire_scatter(m):
        pltpu.async_copy(sstg.at[m], acc_sp.at[dstb.at[m]], sem_sc[m],
                         add=True)
        pltpu.async_copy(sbuf.at[m], den_sp.at[dstb.at[m]], sem_sc[m],
                         add=True)

    def wait_scatter(m):
        pltpu.make_async_copy(sstg.at[m], acc_sp.at[dstb.at[m]],
                              sem_sc[m]).wait()
        pltpu.make_async_copy(sbuf.at[m], den_sp.at[dstb.at[m]],
                              sem_sc[m]).wait()

    pltpu.sync_copy(src_hbm.at[sid, 0], srcb.at[0])
    pltpu.sync_copy(src_hbm.at[sid, 1], srcb.at[1])
    pltpu.sync_copy(src_hbm.at[sid, 2], srcb.at[2])
    for m in range(NB):
        pltpu.sync_copy(dst_hbm.at[sid, m], dstb.at[m])
        pltpu.sync_copy(s_hbm.at[sid, m], sbuf.at[m])
    fire_gather(0)
    fire_gather(1)

    def quad(p, _):
        for b in range(NB):
            c = NB * p + b
            m1 = (b + 1) % NB
            m2 = (b + 2) % NB
            m3 = (b + 3) % NB

            @pl.when(c >= 2)
            def _():
                wait_scatter(m2)

            @pl.when(jnp.logical_and(c >= 2, c + 2 < C_CHUNKS))
            def _():
                fire_ds(c + 2, m2)

            @pl.when(c + 3 < C_CHUNKS)
            def _():
                fire_src(c + 3, m3)

            @pl.when(jnp.logical_and(c >= 1, c + 2 < C_CHUNKS))
            def _():
                wait_src(c + 2, m2)

            @pl.when(c + 2 < C_CHUNKS)
            def _():
                fire_gather(m2)

            wait_gather(b)

            @pl.when(c >= 4)
            def _():
                wait_ds(c, b)
            _unpack_scale_rows(gbuf, sstg, b, sbuf, K)
            fire_scatter(b)
        return 0

    lax.fori_loop(0, C_CHUNKS // NB, quad, 0)
    wait_scatter((C_CHUNKS - 2) % NB)
    wait_scatter((C_CHUNKS - 1) % NB)
    plsc.subcore_barrier()

    def norm_write(c, n_rows):
        pltpu.sync_copy(acc_sp.at[pl.ds(c * K, n_rows), :],
                        sstg.at[0, pl.ds(0, n_rows), :])
        pltpu.sync_copy(den_sp.at[pl.ds(c * K, n_rows)],
                        sbuf.at[0, pl.ds(0, n_rows)])
        for q in range(n_rows // L):
            sbuf[0, pl.ds(q * L, L)] = 1.0 / sbuf[0, pl.ds(q * L, L)]
        _scale_rows(sstg, 0, sbuf, n_rows)

        @pl.when(cid == 0)
        def _():
            pltpu.sync_copy(sstg.at[0, pl.ds(0, n_rows), :],
                            out_hbm.at[pl.ds(c * K, n_rows), pl.ds(0, DH)])

        @pl.when(cid == 1)
        def _():
            pltpu.sync_copy(sstg.at[0, pl.ds(0, n_rows), :],
                            out_hbm.at[pl.ds(c * K, n_rows), pl.ds(DH, DH)])

    for j in range(-(-ROW_CHUNKS // NS)):
        c = sid + NS * j

        @pl.when(c < ROW_CHUNKS - 1)
        def _():
            norm_write(c, K)

        @pl.when(c == ROW_CHUNKS - 1)
        def _():
            norm_write(jnp.int32(ROW_CHUNKS - 1), LAST_ROWS)


_agg_stage = functools.partial(
    pl.kernel,
    out_type=jax.ShapeDtypeStruct((N, D_OUT), jnp.float32),
    mesh=plsc.VectorSubcoreMesh(core_axis_name="c", subcore_axis_name="s",
                                num_cores=NC, num_subcores=NS),
    compiler_params=pltpu.CompilerParams(needs_layout_passes=False,
                                         use_tc_tiling_on_sc=False),
    scratch_types=(
        [
            pltpu.VMEM((NB, K), jnp.int32),
            pltpu.VMEM((NB, K), jnp.int32),
            pltpu.VMEM((NB, K), jnp.float32),
            pltpu.VMEM((NB, K, DH // 2), jnp.int32),
            pltpu.VMEM((NB, K, DH), jnp.float32),
            pltpu.VMEM_SHARED((N, DH), jnp.float32),
            pltpu.VMEM_SHARED((N,), jnp.float32),
        ]
        + [pltpu.SemaphoreType.DMA] * 16
    ),
)(_agg_body)


def kernel(node_feature, edge_index, W, att_src, att_dst):
    attp = jnp.concatenate(
        [att_src[:, None], att_dst[:, None],
         jnp.zeros((D_IN, 6), jnp.float32)], axis=1)
    x0, x1, att = _dense_stage(node_feature, W, attp)

    ei = edge_index.astype(jnp.int32)
    loops = jnp.arange(N, dtype=jnp.int32)
    src = jnp.concatenate([ei[0], loops])
    dst = jnp.concatenate([ei[1], loops])
    src = jnp.pad(src, (0, E_PAD - ET)).reshape(NS, C_CHUNKS, K)
    dst = jnp.pad(dst, (0, E_PAD - ET)).reshape(NS, C_CHUNKS, K)

    def _pack(xh):
        xbf = xh.astype(jnp.bfloat16).reshape(N, 4, 2, 16)
        xbf = xbf.transpose(0, 1, 3, 2).reshape(N, DH // 2, 2)
        return lax.bitcast_convert_type(xbf, jnp.int32)

    s_pad = _s_stage(att, src, dst)
    return _agg_stage(_pack(x0), _pack(x1), src, dst, s_pad)


# speedup vs baseline: 1.2394x; 1.2383x over previous
import functools

import jax
import jax.numpy as jnp
from jax import lax
from jax.experimental import pallas as pl
from jax.experimental.pallas import tpu as pltpu
from jax.experimental.pallas import tpu_sc as plsc

N = 10000
D_IN = 256
D_OUT = 256
DH = 128
NEG_SLOPE = 0.2

NC = 2
NS = 16
L = 16

ET = 160000 + N
K = 64
NB = 4
_c = -(-(-(-ET // NS)) // K)
C_CHUNKS = -(-_c // NB) * NB
PT = C_CHUNKS * K
E_PAD = PT * NS

ROW_CHUNKS = -(-N // K)
LAST_ROWS = N - (ROW_CHUNKS - 1) * K


def _tc_body(nf_ref, w_ref, attp_ref, x0_ref, x1_ref, att_ref):
    xw = jnp.dot(nf_ref[...], w_ref[...], preferred_element_type=jnp.float32)
    x0_ref[...] = xw[:, :DH]
    x1_ref[...] = xw[:, DH:]
    att_ref[...] = lax.dot_general(
        attp_ref[...], xw, (((0,), (1,)), ((), ())),
        preferred_element_type=jnp.float32)


def _dense_stage(node_feature, W, attp):
    return pl.pallas_call(
        _tc_body,
        out_shape=[
            jax.ShapeDtypeStruct((N, DH), jnp.float32),
            jax.ShapeDtypeStruct((N, DH), jnp.float32),
            jax.ShapeDtypeStruct((8, N), jnp.float32),
        ],
    )(node_feature, W, attp)


def _s_body(att_hbm, src_hbm, dst_hbm, s_hbm, table, idxb, s_all):
    cid = lax.axis_index("c")
    sid = lax.axis_index("s")

    @pl.when(cid == 0)
    def _():
        base = sid * PT
        iota = lax.iota(jnp.int32, L)

        pltpu.sync_copy(att_hbm.at[0], table)
        pltpu.sync_copy(src_hbm.at[sid], idxb)

        def p1(c, _):
            for q in range(K // L):
                si = idxb[c, pl.ds(q * L, L)]
                s_all[c, pl.ds(q * L, L)] = plsc.load_gather(table, [si])
            return 0
        lax.fori_loop(0, C_CHUNKS, p1, 0, unroll=2)

        pltpu.sync_copy(att_hbm.at[1], table)
        pltpu.sync_copy(dst_hbm.at[sid], idxb)

        def p2(c, _):
            for q in range(K // L):
                di = idxb[c, pl.ds(q * L, L)]
                al = s_all[c, pl.ds(q * L, L)] + plsc.load_gather(table, [di])
                al = jnp.where(al > 0, al, al * NEG_SLOPE)
                s = jnp.exp(al)
                pos = base + c * K + q * L + iota
                s_all[c, pl.ds(q * L, L)] = jnp.where(pos < ET, s, 0.0)
            return 0
        lax.fori_loop(0, C_CHUNKS, p2, 0, unroll=2)

        pltpu.sync_copy(s_all, s_hbm.at[sid])


_s_stage = functools.partial(
    pl.kernel,
    out_type=jax.ShapeDtypeStruct((NS, C_CHUNKS, K), jnp.float32),
    mesh=plsc.VectorSubcoreMesh(core_axis_name="c", subcore_axis_name="s",
                                num_cores=NC, num_subcores=NS),
    compiler_params=pltpu.CompilerParams(needs_layout_passes=False),
    scratch_types=[
        pltpu.VMEM((N,), jnp.float32),
        pltpu.VMEM((C_CHUNKS, K), jnp.int32),
        pltpu.VMEM((C_CHUNKS, K), jnp.float32),
    ],
)(_s_body)


def _scale_rows(gbuf, b, sbuf, n_rows):
    bsplat = jnp.full((L,), b, jnp.int32)

    def body(i, _):
        sv = plsc.load_gather(sbuf, [bsplat, jnp.full((L,), i, jnp.int32)])
        for q in range(DH // L):
            gbuf[b, i, pl.ds(q * L, L)] = gbuf[b, i, pl.ds(q * L, L)] * sv
        return 0
    lax.fori_loop(0, n_rows, body, 0, unroll=2)


def _agg_body(x0_hbm, x1_hbm, src_hbm, dst_hbm, s_hbm, out_hbm,
              srcb, dstb, sbuf, gbuf, acc_sp, den_sp,
              sem_g0, sem_g1, sem_g2, sem_g3,
              sem_sc0, sem_sc1, sem_sc2, sem_sc3,
              sem_s0, sem_s1, sem_s2, sem_s3,
              sem_d0, sem_d1, sem_d2, sem_d3):
    cid = lax.axis_index("c")
    sid = lax.axis_index("s")
    sem_g = (sem_g0, sem_g1, sem_g2, sem_g3)
    sem_sc = (sem_sc0, sem_sc1, sem_sc2, sem_sc3)
    sem_s = (sem_s0, sem_s1, sem_s2, sem_s3)
    sem_d = (sem_d0, sem_d1, sem_d2, sem_d3)

    def zrow(i, _):
        for q in range(DH // L):
            gbuf[0, i, pl.ds(q * L, L)] = jnp.zeros((L,), jnp.float32)
        return 0
    lax.fori_loop(0, K, zrow, 0)
    for q in range(K // L):
        sbuf[0, pl.ds(q * L, L)] = jnp.zeros((L,), jnp.float32)
    for j in range(-(-ROW_CHUNKS // NS)):
        c = sid + NS * j

        @pl.when(c < ROW_CHUNKS - 1)
        def _():
            pltpu.sync_copy(gbuf.at[0], acc_sp.at[pl.ds(c * K, K), :])
            pltpu.sync_copy(sbuf.at[0], den_sp.at[pl.ds(c * K, K)])

        @pl.when(c == ROW_CHUNKS - 1)
        def _():
            pltpu.sync_copy(gbuf.at[0, pl.ds(0, LAST_ROWS), :],
                            acc_sp.at[pl.ds((ROW_CHUNKS - 1) * K, LAST_ROWS), :])
            pltpu.sync_copy(sbuf.at[0, pl.ds(0, LAST_ROWS)],
                            den_sp.at[pl.ds((ROW_CHUNKS - 1) * K, LAST_ROWS)])
    plsc.subcore_barrier()

    def fire_src(c, m):
        pltpu.async_copy(src_hbm.at[sid, c], srcb.at[m], sem_s[m])

    def wait_src(c, m):
        pltpu.make_async_copy(src_hbm.at[sid, c], srcb.at[m], sem_s[m]).wait()

    def fire_ds(c, m):
        pltpu.async_copy(dst_hbm.at[sid, c], dstb.at[m], sem_d[m])
        pltpu.async_copy(s_hbm.at[sid, c], sbuf.at[m], sem_d[m])

    def wait_ds(c, m):
        pltpu.make_async_copy(dst_hbm.at[sid, c], dstb.at[m], sem_d[m]).wait()
        pltpu.make_async_copy(s_hbm.at[sid, c], sbuf.at[m], sem_d[m]).wait()

    def fire_gather(m):
        @pl.when(cid == 0)
        def _():
            pltpu.async_copy(x0_hbm.at[srcb.at[m]], gbuf.at[m], sem_g[m])

        @pl.when(cid == 1)
        def _():
            pltpu.async_copy(x1_hbm.at[srcb.at[m]], gbuf.at[m], sem_g[m])

    def wait_gather(m):
        pltpu.make_async_copy(x0_hbm.at[srcb.at[m]], gbuf.at[m],
                              sem_g[m]).wait()

    def fire_scatter(m):
        pltpu.async_copy(gbuf.at[m], acc_sp.at[dstb.at[m]], sem_sc[m],
                         add=True)
        pltpu.async_copy(sbuf.at[m], den_sp.at[dstb.at[m]], sem_sc[m],
                         add=True)

    def wait_scatter(m):
        pltpu.make_async_copy(gbuf.at[m], acc_sp.at[dstb.at[m]],
                              sem_sc[m]).wait()
        pltpu.make_async_copy(sbuf.at[m], den_sp.at[dstb.at[m]],
                              sem_sc[m]).wait()

    pltpu.sync_copy(src_hbm.at[sid, 0], srcb.at[0])
    pltpu.sync_copy(src_hbm.at[sid, 1], srcb.at[1])
    pltpu.sync_copy(src_hbm.at[sid, 2], srcb.at[2])
    for m in range(NB):
        pltpu.sync_copy(dst_hbm.at[sid, m], dstb.at[m])
        pltpu.sync_copy(s_hbm.at[sid, m], sbuf.at[m])
    fire_gather(0)
    fire_gather(1)

    def quad(p, _):
        for b in range(NB):
            c = NB * p + b
            m1 = (b + 1) % NB
            m2 = (b + 2) % NB
            m3 = (b + 3) % NB

            @pl.when(c >= 2)
            def _():
                wait_scatter(m2)

            @pl.when(jnp.logical_and(c >= 2, c + 2 < C_CHUNKS))
            def _():
                fire_ds(c + 2, m2)

            @pl.when(c + 3 < C_CHUNKS)
            def _():
                fire_src(c + 3, m3)

            @pl.when(jnp.logical_and(c >= 1, c + 2 < C_CHUNKS))
            def _():
                wait_src(c + 2, m2)

            @pl.when(c + 2 < C_CHUNKS)
            def _():
                fire_gather(m2)

            wait_gather(b)

            @pl.when(c >= 4)
            def _():
                wait_ds(c, b)
            _scale_rows(gbuf, b, sbuf, K)
            fire_scatter(b)
        return 0

    lax.fori_loop(0, C_CHUNKS // NB, quad, 0)
    wait_scatter((C_CHUNKS - 2) % NB)
    wait_scatter((C_CHUNKS - 1) % NB)
    plsc.subcore_barrier()

    def norm_write(c, n_rows):
        pltpu.sync_copy(acc_sp.at[pl.ds(c * K, n_rows), :],
                        gbuf.at[0, pl.ds(0, n_rows), :])
        pltpu.sync_copy(den_sp.at[pl.ds(c * K, n_rows)],
                        sbuf.at[0, pl.ds(0, n_rows)])
        for q in range(n_rows // L):
            sbuf[0, pl.ds(q * L, L)] = 1.0 / sbuf[0, pl.ds(q * L, L)]
        _scale_rows(gbuf, 0, sbuf, n_rows)

        @pl.when(cid == 0)
        def _():
            pltpu.sync_copy(gbuf.at[0, pl.ds(0, n_rows), :],
                            out_hbm.at[pl.ds(c * K, n_rows), pl.ds(0, DH)])

        @pl.when(cid == 1)
        def _():
            pltpu.sync_copy(gbuf.at[0, pl.ds(0, n_rows), :],
                            out_hbm.at[pl.ds(c * K, n_rows), pl.ds(DH, DH)])

    for j in range(-(-ROW_CHUNKS // NS)):
        c = sid + NS * j

        @pl.when(c < ROW_CHUNKS - 1)
        def _():
            norm_write(c, K)

        @pl.when(c == ROW_CHUNKS - 1)
        def _():
            norm_write(jnp.int32(ROW_CHUNKS - 1), LAST_ROWS)


_agg_stage = functools.partial(
    pl.kernel,
    out_type=jax.ShapeDtypeStruct((N, D_OUT), jnp.float32),
    mesh=plsc.VectorSubcoreMesh(core_axis_name="c", subcore_axis_name="s",
                                num_cores=NC, num_subcores=NS),
    compiler_params=pltpu.CompilerParams(needs_layout_passes=False),
    scratch_types=(
        [
            pltpu.VMEM((NB, K), jnp.int32),
            pltpu.VMEM((NB, K), jnp.int32),
            pltpu.VMEM((NB, K), jnp.float32),
            pltpu.VMEM((NB, K, DH), jnp.float32),
            pltpu.VMEM_SHARED((N, DH), jnp.float32),
            pltpu.VMEM_SHARED((N,), jnp.float32),
        ]
        + [pltpu.SemaphoreType.DMA] * 16
    ),
)(_agg_body)


def kernel(node_feature, edge_index, W, att_src, att_dst):
    attp = jnp.concatenate(
        [att_src[:, None], att_dst[:, None],
         jnp.zeros((D_IN, 6), jnp.float32)], axis=1)
    x0, x1, att = _dense_stage(node_feature, W, attp)

    ei = edge_index.astype(jnp.int32)
    loops = jnp.arange(N, dtype=jnp.int32)
    src = jnp.concatenate([ei[0], loops])
    dst = jnp.concatenate([ei[1], loops])
    src = jnp.pad(src, (0, E_PAD - ET)).reshape(NS, C_CHUNKS, K)
    dst = jnp.pad(dst, (0, E_PAD - ET)).reshape(NS, C_CHUNKS, K)

    s_pad = _s_stage(att, src, dst)
    return _agg_stage(x0, x1, src, dst, s_pad)
